# baseline (speedup 1.0000x reference)
import jax
import jax.numpy as jnp
from jax import lax
from jax.experimental import pallas as pl
from jax.experimental.pallas import tpu as pltpu
from jax.experimental.pallas import tpu_sc as plsc

VOCAB = 1000000
DIM = 64
BATCH = 16384

_INFO = plsc.get_sparse_core_info()
_NC = _INFO.num_cores
_NS = _INFO.num_subcores
_NW = _NC * _NS
_L = _INFO.num_lanes

_NCOL = VOCAB // 128 + 1
_WPT = _NW // 2
_Q = 490
_RING = 10
_NSTEP = (_Q + _RING - 1) // _RING
_IC = 512
_SEL = 1296
_NVEC = _SEL // _L
_EROWS = BATCH + 128
_DUMP = BATCH + 8


def _splat(x, i):
    return x.at[jnp.full((_L,), i, jnp.int32)].get(mode="promise_in_bounds")


def _phase1(cw_hbm, xw_hbm, ctab_hbm, xtab_hbm, ctail_hbm, xtail_hbm,
            e1_hbm, e2_hbm,
            idx_v, selv_v, selp_v, selv2_v, selp2_v,
            blk_v, stg_v, dix_v, tail_v, sems):
    wid = lax.axis_index("s") * _NC + lax.axis_index("c")
    k = wid & (_WPT - 1)
    lo_col = k * _Q
    n_my = jnp.minimum(_Q, jnp.maximum(_NCOL - lo_col, 0))
    n_full = jnp.minimum(_Q, jnp.maximum(_NCOL - 1 - lo_col, 0))

    iota = lax.iota(jnp.int32, _L)
    dq = [q * _L + iota for q in range(DIM // _L)]

    def body(words_hbm, tab_hbm, tail_hbm, out_hbm):
        def sel_chunk(c, off0):
            pltpu.sync_copy(words_hbm.at[pl.ds(c * _IC, _IC)], idx_v)

            def sel_step(i, off, c=c):
                v16 = idx_v[pl.ds(i * _L, _L)]
                col = lax.shift_right_logical(v16, 7) - lo_col
                m = (col >= 0) & (col < n_my)
                plsc.store_compressed(selv_v.at[pl.ds(off, _L)], v16, mask=m)
                plsc.store_compressed(selp_v.at[pl.ds(off, _L)],
                                      c * _IC + i * _L + iota, mask=m)
                return off + plsc.all_reduce_population_count(m)[0]

            return lax.fori_loop(0, _IC // _L, sel_step, off0, unroll=False)

        nsel = lax.fori_loop(0, BATCH // _IC, sel_chunk, 0, unroll=False)

        bufs = [(selv_v, selp_v), (selv2_v, selp2_v)]
        for b in range(9):
            src_v, src_p = bufs[b % 2]
            dst_v, dst_p = bufs[(b + 1) % 2]

            def half(bitval, off0, b=b, src_v=src_v, src_p=src_p,
                     dst_v=dst_v, dst_p=dst_p):
                def pass_step(i, off):
                    v16 = src_v[pl.ds(i * _L, _L)]
                    p16 = src_p[pl.ds(i * _L, _L)]
                    key = (lax.shift_right_logical(v16, 7) - lo_col)
                    bit = lax.shift_right_logical(key, b) & 1
                    m = ((i * _L + iota) < nsel) & (bit == bitval)
                    plsc.store_compressed(dst_v.at[pl.ds(off, _L)], v16, mask=m)
                    plsc.store_compressed(dst_p.at[pl.ds(off, _L)], p16, mask=m)
                    return off + plsc.all_reduce_population_count(m)[0]

                return lax.fori_loop(0, _NVEC, pass_step, off0, unroll=False)

            half(1, half(0, 0))
        sorted_v, sorted_p = bufs[1]

        for q in range(_L // 2):
            dix_v[0, pl.ds(q * _L, _L)] = jnp.full((_L,), _DUMP, jnp.int32)
            dix_v[1, pl.ds(q * _L, _L)] = jnp.full((_L,), _DUMP, jnp.int32)

        def fire(cl, slot):
            @pl.when(cl < n_full)
            def _():
                base = pl.multiple_of((cl + lo_col) * 128, 128)
                pltpu.async_copy(
                    tab_hbm.at[:, pl.ds(base, 128)],
                    blk_v.at[slot], sems.at[slot])

        def drain(cl, slot):
            @pl.when(cl < n_full)
            def _():
                pltpu.make_async_copy(
                    tab_hbm.at[:, pl.ds(0, 128)], blk_v.at[slot],
                    sems.at[slot]).wait()

        def process(cl, slot, state):
            ptr, cnt = state

            def cond(s):
                p, _c = s
                vvec = sorted_v[pl.ds((p // _L) * _L, _L)]
                vs = _splat(vvec, p % _L)
                incol = lax.shift_right_logical(vs[0], 7) - lo_col == cl
                return (p < nsel) & incol & (cl < n_full)

            def mbody(s):
                p, c = s
                vs = _splat(sorted_v[pl.ds((p // _L) * _L, _L)], p % _L)
                ps = _splat(sorted_p[pl.ds((p // _L) * _L, _L)], p % _L)
                lane = vs & 127
                row = c & 255
                for q in range(DIM // _L):
                    g = plsc.load_gather(blk_v.at[slot], [dq[q], lane])
                    stg_v[row, pl.ds(q * _L, _L)] = g
                plsc.store_scatter(
                    dix_v,
                    [jnp.full((_L,), lax.shift_right_logical(c, 7) & 1,
                              jnp.int32),
                     jnp.full((_L,), c & 127, jnp.int32)],
                    ps, mask=iota == 0)
                return p + 1, c + 1

            return lax.while_loop(cond, mbody, (ptr, cnt))

        def flush(state):
            cnt, flushed = state

            @pl.when(cnt - flushed >= 128)
            def _():
                half = lax.shift_right_logical(flushed, 7) & 1
                pltpu.sync_copy(stg_v.at[pl.ds(half * 128, 128)],
                                out_hbm.at[dix_v.at[half]])
                for q in range(_L // 2):
                    dix_v[half, pl.ds(q * _L, _L)] = jnp.full(
                        (_L,), _DUMP, jnp.int32)

            return jnp.where(cnt - flushed >= 128, flushed + 128, flushed)

        for slot in range(_RING):
            fire(slot, slot)

        def step(s, carry):
            ptr, cnt, flushed = carry
            for slot in range(_RING):
                cl = _RING * s + slot
                drain(cl, slot)
                ptr, cnt = process(cl, slot, (ptr, cnt))

                @pl.when(cl + _RING < _Q)
                def _(cl=cl, slot=slot):
                    fire(cl + _RING, slot)

                flushed = flush((cnt, flushed))
            return ptr, cnt, flushed

        ptr, cnt, flushed = lax.fori_loop(0, _NSTEP, step, (0, 0, 0),
                                          unroll=False)

        @pl.when(n_my > n_full)
        def _():
            pltpu.sync_copy(tail_hbm, tail_v)

        def scond(s):
            p, _c = s
            vs = _splat(sorted_v[pl.ds((p // _L) * _L, _L)], p % _L)
            return (p < nsel) & (
                lax.shift_right_logical(vs[0], 7) - lo_col == n_full)

        def sbody(s):
            p, c = s
            vs = _splat(sorted_v[pl.ds((p // _L) * _L, _L)], p % _L)
            ps = _splat(sorted_p[pl.ds((p // _L) * _L, _L)], p % _L)
            lane = vs & 127
            row = c & 255
            for q in range(DIM // _L):
                g = plsc.load_gather(tail_v, [dq[q] * 64 + lane])
                stg_v[row, pl.ds(q * _L, _L)] = g
            plsc.store_scatter(
                dix_v,
                [jnp.full((_L,), lax.shift_right_logical(c, 7) & 1,
                          jnp.int32),
                 jnp.full((_L,), c & 127, jnp.int32)],
                ps, mask=iota == 0)
            return p + 1, c + 1

        ptr, cnt = lax.while_loop(scond, sbody, (ptr, cnt))
        flushed = flush((cnt, flushed))

        pltpu.sync_copy(stg_v.at[pl.ds(0, 128)], out_hbm.at[dix_v.at[0]])
        pltpu.sync_copy(stg_v.at[pl.ds(128, 128)], out_hbm.at[dix_v.at[1]])

    @pl.when(wid < _WPT)
    def _():
        body(cw_hbm, ctab_hbm, ctail_hbm, e1_hbm)

    @pl.when(wid >= _WPT)
    def _():
        body(xw_hbm, xtab_hbm, xtail_hbm, e2_hbm)


_RW = BATCH // _NW
_C2 = 128


def _phase2(e1_hbm, e2_hbm, out_hbm, c_v, x_v, out_v, sems):
    wid = lax.axis_index("s") * _NC + lax.axis_index("c")
    base = wid * _RW
    iota = lax.iota(jnp.int32, _L)
    pend = {}

    def fire2(p):
        b = p & 1
        pend[p] = (
            pltpu.async_copy(e1_hbm.at[pl.ds(base + p * _C2, _C2), :],
                             c_v.at[b], sems.at[b]),
            pltpu.async_copy(e2_hbm.at[pl.ds(base + p * _C2, _C2), :],
                             x_v.at[b], sems.at[b]),
        )

    fire2(0)
    for p in range(_RW // _C2):
        b = p & 1
        if p + 1 < _RW // _C2:
            fire2(p + 1)
        ca, cb = pend.pop(p)
        ca.wait()
        cb.wait()

        def body(g, carry, p=p, b=b):
            rows = g * _L + iota

            def dstep(d, acc):
                cv = plsc.load_gather(c_v.at[b], [rows, jnp.full((_L,), d,
                                                                 jnp.int32)])
                xv = plsc.load_gather(x_v.at[b], [rows, jnp.full((_L,), d,
                                                                 jnp.int32)])
                return acc + cv * xv

            tot = lax.fori_loop(0, DIM, dstep, jnp.zeros((_L,), jnp.float32),
                                unroll=8)
            out_v[pl.ds(p * _C2 + g * _L, _L)] = tot
            return carry

        lax.fori_loop(0, _C2 // _L, body, 0, unroll=False)

    pltpu.sync_copy(out_v, out_hbm.at[pl.ds(base, _RW)])


@jax.jit
def kernel(center_words, context_words, center_table, context_table):
    cw = center_words.astype(jnp.int32)
    xw = context_words.astype(jnp.int32)
    mesh = plsc.VectorSubcoreMesh(core_axis_name="c", subcore_axis_name="s")
    run1 = pl.kernel(
        _phase1,
        out_type=(jax.ShapeDtypeStruct((_EROWS, 128), jnp.float32),
                  jax.ShapeDtypeStruct((_EROWS, 128), jnp.float32)),
        mesh=mesh,
        scratch_types=[
            pltpu.VMEM((_IC,), jnp.int32),
            pltpu.VMEM((_SEL + _L,), jnp.int32),
            pltpu.VMEM((_SEL + _L,), jnp.int32),
            pltpu.VMEM((_SEL + _L,), jnp.int32),
            pltpu.VMEM((_SEL + _L,), jnp.int32),
            pltpu.VMEM((_RING, 64, 128), jnp.float32),
            pltpu.VMEM((256, 128), jnp.float32),
            pltpu.VMEM((2, 128), jnp.int32),
            pltpu.VMEM((64 * 64,), jnp.float32),
            pltpu.SemaphoreType.DMA((_RING,)),
        ],
        compiler_params=pltpu.CompilerParams(needs_layout_passes=False),
    )
    ctail = center_table.T[:, VOCAB - 64:].reshape(64 * 64)
    xtail = context_table.T[:, VOCAB - 64:].reshape(64 * 64)
    e1, e2 = run1(cw, xw, center_table.T, context_table.T, ctail, xtail)
    run2 = pl.kernel(
        _phase2,
        out_type=jax.ShapeDtypeStruct((BATCH,), jnp.float32),
        mesh=mesh,
        scratch_types=[
            pltpu.VMEM((2, _C2, 128), jnp.float32),
            pltpu.VMEM((2, _C2, 128), jnp.float32),
            pltpu.VMEM((_RW,), jnp.float32),
            pltpu.SemaphoreType.DMA((2,)),
        ],
        compiler_params=pltpu.CompilerParams(needs_layout_passes=False),
    )
    return run2(e1, e2)


# speedup vs baseline: 1.0767x; 1.0767x over previous
import jax
import jax.numpy as jnp
from jax import lax
from jax.experimental import pallas as pl
from jax.experimental.pallas import tpu as pltpu
from jax.experimental.pallas import tpu_sc as plsc

VOCAB = 1000000
DIM = 64
BATCH = 16384

_INFO = plsc.get_sparse_core_info()
_NC = _INFO.num_cores
_NS = _INFO.num_subcores
_NW = _NC * _NS
_L = _INFO.num_lanes

_NCOL = VOCAB // 128 + 1
_WPT = _NW // 2
_Q = 489
_RING = 8
_NSTEP = (_Q + _RING - 1) // _RING
_SEL = 1296
_NVEC = _SEL // _L
_EROWS = BATCH + 128
_DUMP = BATCH + 8


def _splat(x, i):
    return x.at[jnp.full((_L,), i, jnp.int32)].get(mode="promise_in_bounds")


def _phase1(cw_hbm, xw_hbm, ctab_hbm, xtab_hbm, ctail_hbm, xtail_hbm,
            e1_hbm, e2_hbm,
            idx_v, selv_v, selp_v, selv2_v, selp2_v,
            blk_v, stg_v, dix_v, tail_v, sems):
    wid = lax.axis_index("s") * _NC + lax.axis_index("c")
    k = wid & (_WPT - 1)
    lo_col = k * _Q
    n_my = jnp.minimum(_Q, jnp.maximum(_NCOL - lo_col, 0))
    n_full = jnp.minimum(_Q, jnp.maximum(_NCOL - 1 - lo_col, 0))

    iota = lax.iota(jnp.int32, _L)
    dq = [q * _L + iota for q in range(DIM // _L)]

    def body(words_hbm, tab_hbm, tail_hbm, out_hbm):
        pltpu.sync_copy(words_hbm, idx_v)

        def sel_step(i, off):
            v16 = idx_v[pl.ds(i * _L, _L)]
            col = lax.shift_right_logical(v16, 7) - lo_col
            m = (col >= 0) & (col < n_my)
            plsc.store_compressed(selv_v.at[pl.ds(off, _L)], v16, mask=m)
            plsc.store_compressed(selp_v.at[pl.ds(off, _L)], i * _L + iota,
                                  mask=m)
            return off + plsc.all_reduce_population_count(m)[0]

        nsel = lax.fori_loop(0, BATCH // _L, sel_step, 0, unroll=False)

        bufs = [(selv_v, selp_v), (selv2_v, selp2_v)]
        for b in range(9):
            src_v, src_p = bufs[b % 2]
            dst_v, dst_p = bufs[(b + 1) % 2]

            def half(bitval, off0, b=b, src_v=src_v, src_p=src_p,
                     dst_v=dst_v, dst_p=dst_p):
                def pass_step(i, off):
                    v16 = src_v[pl.ds(i * _L, _L)]
                    p16 = src_p[pl.ds(i * _L, _L)]
                    key = (lax.shift_right_logical(v16, 7) - lo_col)
                    bit = lax.shift_right_logical(key, b) & 1
                    m = ((i * _L + iota) < nsel) & (bit == bitval)
                    plsc.store_compressed(dst_v.at[pl.ds(off, _L)], v16, mask=m)
                    plsc.store_compressed(dst_p.at[pl.ds(off, _L)], p16, mask=m)
                    return off + plsc.all_reduce_population_count(m)[0]

                return lax.fori_loop(0, _NVEC, pass_step, off0, unroll=False)

            half(1, half(0, 0))
        sorted_v, sorted_p = bufs[1]

        for q in range(_L // 2):
            dix_v[0, pl.ds(q * _L, _L)] = jnp.full((_L,), _DUMP, jnp.int32)
            dix_v[1, pl.ds(q * _L, _L)] = jnp.full((_L,), _DUMP, jnp.int32)

        def fire(cl, slot):
            @pl.when(cl < n_full)
            def _():
                base = pl.multiple_of((cl + lo_col) * 128, 128)
                pltpu.async_copy(
                    tab_hbm.at[:, pl.ds(base, 128)],
                    blk_v.at[slot], sems.at[slot])

        def drain(cl, slot):
            @pl.when(cl < n_full)
            def _():
                pltpu.make_async_copy(
                    tab_hbm.at[:, pl.ds(0, 128)], blk_v.at[slot],
                    sems.at[slot]).wait()

        def process(cl, slot, state):
            ptr, cnt = state

            def cond(s):
                p, _c = s
                vvec = sorted_v[pl.ds((p // _L) * _L, _L)]
                vs = _splat(vvec, p % _L)
                incol = lax.shift_right_logical(vs[0], 7) - lo_col == cl
                return (p < nsel) & incol & (cl < n_full)

            def mbody(s):
                p, c = s
                vs = _splat(sorted_v[pl.ds((p // _L) * _L, _L)], p % _L)
                ps = _splat(sorted_p[pl.ds((p // _L) * _L, _L)], p % _L)
                lane = vs & 127
                row = c & 255
                for q in range(DIM // _L):
                    g = plsc.load_gather(blk_v.at[slot], [dq[q], lane])
                    stg_v[row, pl.ds(q * _L, _L)] = g
                plsc.store_scatter(
                    dix_v,
                    [jnp.full((_L,), lax.shift_right_logical(c, 7) & 1,
                              jnp.int32),
                     jnp.full((_L,), c & 127, jnp.int32)],
                    ps, mask=iota == 0)
                return p + 1, c + 1

            return lax.while_loop(cond, mbody, (ptr, cnt))

        def flush(state):
            cnt, flushed = state

            @pl.when(cnt - flushed >= 128)
            def _():
                half = lax.shift_right_logical(flushed, 7) & 1
                pltpu.sync_copy(stg_v.at[pl.ds(half * 128, 128)],
                                out_hbm.at[dix_v.at[half]])
                for q in range(_L // 2):
                    dix_v[half, pl.ds(q * _L, _L)] = jnp.full(
                        (_L,), _DUMP, jnp.int32)

            return jnp.where(cnt - flushed >= 128, flushed + 128, flushed)

        for slot in range(_RING):
            fire(slot, slot)

        def step(s, carry):
            ptr, cnt, flushed = carry
            for slot in range(_RING):
                cl = _RING * s + slot
                drain(cl, slot)
                ptr, cnt = process(cl, slot, (ptr, cnt))

                @pl.when(cl + _RING < _Q)
                def _(cl=cl, slot=slot):
                    fire(cl + _RING, slot)

                flushed = flush((cnt, flushed))
            return ptr, cnt, flushed

        ptr, cnt, flushed = lax.fori_loop(0, _NSTEP, step, (0, 0, 0),
                                          unroll=False)

        @pl.when(n_my > n_full)
        def _():
            pltpu.sync_copy(tail_hbm, tail_v)

        def scond(s):
            p, _c = s
            vs = _splat(sorted_v[pl.ds((p // _L) * _L, _L)], p % _L)
            return (p < nsel) & (
                lax.shift_right_logical(vs[0], 7) - lo_col == n_full)

        def sbody(s):
            p, c = s
            vs = _splat(sorted_v[pl.ds((p // _L) * _L, _L)], p % _L)
            ps = _splat(sorted_p[pl.ds((p // _L) * _L, _L)], p % _L)
            lane = vs & 127
            row = c & 255
            for q in range(DIM // _L):
                g = plsc.load_gather(tail_v, [dq[q] * 64 + lane])
                stg_v[row, pl.ds(q * _L, _L)] = g
            plsc.store_scatter(
                dix_v,
                [jnp.full((_L,), lax.shift_right_logical(c, 7) & 1,
                          jnp.int32),
                 jnp.full((_L,), c & 127, jnp.int32)],
                ps, mask=iota == 0)
            return p + 1, c + 1

        ptr, cnt = lax.while_loop(scond, sbody, (ptr, cnt))
        flushed = flush((cnt, flushed))

        pltpu.sync_copy(stg_v.at[pl.ds(0, 128)], out_hbm.at[dix_v.at[0]])
        pltpu.sync_copy(stg_v.at[pl.ds(128, 128)], out_hbm.at[dix_v.at[1]])

    @pl.when(wid < _WPT)
    def _():
        body(cw_hbm, ctab_hbm, ctail_hbm, e1_hbm)

    @pl.when(wid >= _WPT)
    def _():
        body(xw_hbm, xtab_hbm, xtail_hbm, e2_hbm)


_RW = BATCH // _NW
_C2 = 128


def _phase2(e1_hbm, e2_hbm, out_hbm, c_v, x_v, out_v, sems):
    wid = lax.axis_index("s") * _NC + lax.axis_index("c")
    base = wid * _RW
    iota = lax.iota(jnp.int32, _L)
    pend = {}

    def fire2(p):
        b = p & 1
        pend[p] = (
            pltpu.async_copy(e1_hbm.at[pl.ds(base + p * _C2, _C2), :],
                             c_v.at[b], sems.at[b]),
            pltpu.async_copy(e2_hbm.at[pl.ds(base + p * _C2, _C2), :],
                             x_v.at[b], sems.at[b]),
        )

    fire2(0)
    for p in range(_RW // _C2):
        b = p & 1
        if p + 1 < _RW // _C2:
            fire2(p + 1)
        ca, cb = pend.pop(p)
        ca.wait()
        cb.wait()

        def body(g, carry, p=p, b=b):
            rows = g * _L + iota

            def dstep(d, acc):
                cv = plsc.load_gather(c_v.at[b], [rows, jnp.full((_L,), d,
                                                                 jnp.int32)])
                xv = plsc.load_gather(x_v.at[b], [rows, jnp.full((_L,), d,
                                                                 jnp.int32)])
                return acc + cv * xv

            tot = lax.fori_loop(0, DIM, dstep, jnp.zeros((_L,), jnp.float32),
                                unroll=8)
            out_v[pl.ds(p * _C2 + g * _L, _L)] = tot
            return carry

        lax.fori_loop(0, _C2 // _L, body, 0, unroll=False)

    pltpu.sync_copy(out_v, out_hbm.at[pl.ds(base, _RW)])


@jax.jit
def kernel(center_words, context_words, center_table, context_table):
    cw = center_words.astype(jnp.int32)
    xw = context_words.astype(jnp.int32)
    mesh = plsc.VectorSubcoreMesh(core_axis_name="c", subcore_axis_name="s")
    run1 = pl.kernel(
        _phase1,
        out_type=(jax.ShapeDtypeStruct((_EROWS, 128), jnp.float32),
                  jax.ShapeDtypeStruct((_EROWS, 128), jnp.float32)),
        mesh=mesh,
        scratch_types=[
            pltpu.VMEM((BATCH,), jnp.int32),
            pltpu.VMEM((_SEL + _L,), jnp.int32),
            pltpu.VMEM((_SEL + _L,), jnp.int32),
            pltpu.VMEM((_SEL + _L,), jnp.int32),
            pltpu.VMEM((_SEL + _L,), jnp.int32),
            pltpu.VMEM((_RING, 64, 128), jnp.float32),
            pltpu.VMEM((256, 128), jnp.float32),
            pltpu.VMEM((2, 128), jnp.int32),
            pltpu.VMEM((64 * 64,), jnp.float32),
            pltpu.SemaphoreType.DMA((_RING,)),
        ],
        compiler_params=pltpu.CompilerParams(needs_layout_passes=False),
    )
    ctail = center_table.T[:, VOCAB - 64:].reshape(64 * 64)
    xtail = context_table.T[:, VOCAB - 64:].reshape(64 * 64)
    e1, e2 = run1(cw, xw, center_table.T, context_table.T, ctail, xtail)
    run2 = pl.kernel(
        _phase2,
        out_type=jax.ShapeDtypeStruct((BATCH,), jnp.float32),
        mesh=mesh,
        scratch_types=[
            pltpu.VMEM((2, _C2, 128), jnp.float32),
            pltpu.VMEM((2, _C2, 128), jnp.float32),
            pltpu.VMEM((_RW,), jnp.float32),
            pltpu.SemaphoreType.DMA((2,)),
        ],
        compiler_params=pltpu.CompilerParams(needs_layout_passes=False),
    )
    return run2(e1, e2)


# speedup vs baseline: 1.2434x; 1.1549x over previous
import jax
import jax.numpy as jnp
from jax import lax
from jax.experimental import pallas as pl
from jax.experimental.pallas import tpu as pltpu
from jax.experimental.pallas import tpu_sc as plsc

VOCAB = 1000000
DIM = 64
BATCH = 16384

_INFO = plsc.get_sparse_core_info()
_NC = _INFO.num_cores
_NS = _INFO.num_subcores
_NW = _NC * _NS
_L = _INFO.num_lanes

_NCOL = VOCAB // 128 + 1
_WPT = _NW // 2
_Q = 489
_RING = 10
_NSTEP = (_Q + _RING - 1) // _RING
_SEL = 1296
_NVEC = _SEL // _L
_EROWS = BATCH + 128
_DUMP = BATCH + 8


def _splat(x, i):
    return x.at[jnp.full((_L,), i, jnp.int32)].get(mode="promise_in_bounds")


def _phase1(cw_hbm, xw_hbm, ctab_hbm, xtab_hbm, ctail_hbm, xtail_hbm,
            e1_hbm, e2_hbm,
            idx_v, selv_v, selp_v, selv2_v, selp2_v,
            blk_v, stg_v, dix_v, tail_v, sems):
    wid = lax.axis_index("s") * _NC + lax.axis_index("c")
    k = wid & (_WPT - 1)
    lo_col = k * _Q
    n_my = jnp.minimum(_Q, jnp.maximum(_NCOL - lo_col, 0))
    n_full = jnp.minimum(_Q, jnp.maximum(_NCOL - 1 - lo_col, 0))

    iota = lax.iota(jnp.int32, _L)
    dq = [q * _L + iota for q in range(DIM // _L)]

    def body(words_hbm, tab_hbm, tail_hbm, out_hbm):
        pltpu.sync_copy(words_hbm, idx_v)

        def sel_step(i, off):
            v16 = idx_v[pl.ds(i * _L, _L)]
            col = lax.shift_right_logical(v16, 7) - lo_col
            m = (col >= 0) & (col < n_my)
            plsc.store_compressed(selv_v.at[pl.ds(off, _L)], v16, mask=m)
            plsc.store_compressed(selp_v.at[pl.ds(off, _L)], i * _L + iota,
                                  mask=m)
            return off + plsc.all_reduce_population_count(m)[0]

        nsel = lax.fori_loop(0, BATCH // _L, sel_step, 0, unroll=False)

        bufs = [(selv_v, selp_v), (selv2_v, selp2_v)]
        for b in range(9):
            src_v, src_p = bufs[b % 2]
            dst_v, dst_p = bufs[(b + 1) % 2]

            def half(bitval, off0, b=b, src_v=src_v, src_p=src_p,
                     dst_v=dst_v, dst_p=dst_p):
                def pass_step(i, off):
                    v16 = src_v[pl.ds(i * _L, _L)]
                    p16 = src_p[pl.ds(i * _L, _L)]
                    key = (lax.shift_right_logical(v16, 7) - lo_col)
                    bit = lax.shift_right_logical(key, b) & 1
                    m = ((i * _L + iota) < nsel) & (bit == bitval)
                    plsc.store_compressed(dst_v.at[pl.ds(off, _L)], v16, mask=m)
                    plsc.store_compressed(dst_p.at[pl.ds(off, _L)], p16, mask=m)
                    return off + plsc.all_reduce_population_count(m)[0]

                return lax.fori_loop(0, _NVEC, pass_step, off0, unroll=False)

            half(1, half(0, 0))
        sorted_v, sorted_p = bufs[1]

        for q in range(4):
            dix_v[0, pl.ds(q * _L, _L)] = jnp.full((_L,), _DUMP, jnp.int32)
            dix_v[1, pl.ds(q * _L, _L)] = jnp.full((_L,), _DUMP, jnp.int32)

        def fire(cl, slot):
            @pl.when(cl < n_full)
            def _():
                base = pl.multiple_of((cl + lo_col) * 128, 128)
                pltpu.async_copy(
                    tab_hbm.at[:, pl.ds(base, 128)],
                    blk_v.at[slot], sems.at[slot])

        def drain(cl, slot):
            @pl.when(cl < n_full)
            def _():
                pltpu.make_async_copy(
                    tab_hbm.at[:, pl.ds(0, 128)], blk_v.at[slot],
                    sems.at[slot]).wait()

        def process(cl, slot, state):
            ptr, cnt = state

            def cond(s):
                p, _c = s
                vvec = sorted_v[pl.ds((p // _L) * _L, _L)]
                vs = _splat(vvec, p % _L)
                incol = lax.shift_right_logical(vs[0], 7) - lo_col == cl
                return (p < nsel) & incol & (cl < n_full)

            def mbody(s):
                p, c = s
                vs = _splat(sorted_v[pl.ds((p // _L) * _L, _L)], p % _L)
                ps = _splat(sorted_p[pl.ds((p // _L) * _L, _L)], p % _L)
                lane = vs & 127
                row = c & 127
                for q in range(DIM // _L):
                    g = plsc.load_gather(blk_v.at[slot], [dq[q], lane])
                    stg_v[row, pl.ds(q * _L, _L)] = g
                plsc.store_scatter(
                    dix_v,
                    [jnp.full((_L,), lax.shift_right_logical(c, 6) & 1,
                              jnp.int32),
                     jnp.full((_L,), c & 63, jnp.int32)],
                    ps, mask=iota == 0)
                return p + 1, c + 1

            return lax.while_loop(cond, mbody, (ptr, cnt))

        def flush(state):
            cnt, flushed = state

            @pl.when(cnt - flushed >= 64)
            def _():
                half = lax.shift_right_logical(flushed, 6) & 1
                pltpu.sync_copy(stg_v.at[pl.ds(half * 64, 64)],
                                out_hbm.at[dix_v.at[half]])
                for q in range(4):
                    dix_v[half, pl.ds(q * _L, _L)] = jnp.full(
                        (_L,), _DUMP, jnp.int32)

            return jnp.where(cnt - flushed >= 64, flushed + 64, flushed)

        for slot in range(_RING):
            fire(slot, slot)

        def step(s, carry):
            ptr, cnt, flushed = carry
            for slot in range(_RING):
                cl = _RING * s + slot
                drain(cl, slot)
                ptr, cnt = process(cl, slot, (ptr, cnt))

                @pl.when(cl + _RING < _Q)
                def _(cl=cl, slot=slot):
                    fire(cl + _RING, slot)

                flushed = flush((cnt, flushed))
            return ptr, cnt, flushed

        ptr, cnt, flushed = lax.fori_loop(0, _NSTEP, step, (0, 0, 0),
                                          unroll=False)

        @pl.when(n_my > n_full)
        def _():
            pltpu.sync_copy(tail_hbm, tail_v)

        def scond(s):
            p, _c = s
            vs = _splat(sorted_v[pl.ds((p // _L) * _L, _L)], p % _L)
            return (p < nsel) & (
                lax.shift_right_logical(vs[0], 7) - lo_col == n_full)

        def sbody(s):
            p, c = s
            vs = _splat(sorted_v[pl.ds((p // _L) * _L, _L)], p % _L)
            ps = _splat(sorted_p[pl.ds((p // _L) * _L, _L)], p % _L)
            lane = vs & 127
            row = c & 255
            for q in range(DIM // _L):
                g = plsc.load_gather(tail_v, [dq[q] * 64 + lane])
                stg_v[row, pl.ds(q * _L, _L)] = g
            plsc.store_scatter(
                dix_v,
                [jnp.full((_L,), lax.shift_right_logical(c, 7) & 1,
                          jnp.int32),
                 jnp.full((_L,), c & 127, jnp.int32)],
                ps, mask=iota == 0)
            return p + 1, c + 1

        ptr, cnt = lax.while_loop(scond, sbody, (ptr, cnt))
        flushed = flush((cnt, flushed))

        pltpu.sync_copy(stg_v.at[pl.ds(0, 64)], out_hbm.at[dix_v.at[0]])
        pltpu.sync_copy(stg_v.at[pl.ds(64, 64)], out_hbm.at[dix_v.at[1]])

    @pl.when(wid < _WPT)
    def _():
        body(cw_hbm, ctab_hbm, ctail_hbm, e1_hbm)

    @pl.when(wid >= _WPT)
    def _():
        body(xw_hbm, xtab_hbm, xtail_hbm, e2_hbm)


_RW = BATCH // _NW
_C2 = 128


def _phase2(e1_hbm, e2_hbm, out_hbm, c_v, x_v, out_v, sems):
    wid = lax.axis_index("s") * _NC + lax.axis_index("c")
    base = wid * _RW
    iota = lax.iota(jnp.int32, _L)
    pend = {}

    def fire2(p):
        b = p & 1
        pend[p] = (
            pltpu.async_copy(e1_hbm.at[pl.ds(base + p * _C2, _C2), :],
                             c_v.at[b], sems.at[b]),
            pltpu.async_copy(e2_hbm.at[pl.ds(base + p * _C2, _C2), :],
                             x_v.at[b], sems.at[b]),
        )

    fire2(0)
    for p in range(_RW // _C2):
        b = p & 1
        if p + 1 < _RW // _C2:
            fire2(p + 1)
        ca, cb = pend.pop(p)
        ca.wait()
        cb.wait()

        def body(g, carry, p=p, b=b):
            rows = g * _L + iota

            def dstep(d, acc):
                cv = plsc.load_gather(c_v.at[b], [rows, jnp.full((_L,), d,
                                                                 jnp.int32)])
                xv = plsc.load_gather(x_v.at[b], [rows, jnp.full((_L,), d,
                                                                 jnp.int32)])
                return acc + cv * xv

            tot = lax.fori_loop(0, DIM, dstep, jnp.zeros((_L,), jnp.float32),
                                unroll=8)
            out_v[pl.ds(p * _C2 + g * _L, _L)] = tot
            return carry

        lax.fori_loop(0, _C2 // _L, body, 0, unroll=False)

    pltpu.sync_copy(out_v, out_hbm.at[pl.ds(base, _RW)])


@jax.jit
def kernel(center_words, context_words, center_table, context_table):
    cw = center_words.astype(jnp.int32)
    xw = context_words.astype(jnp.int32)
    mesh = plsc.VectorSubcoreMesh(core_axis_name="c", subcore_axis_name="s")
    run1 = pl.kernel(
        _phase1,
        out_type=(jax.ShapeDtypeStruct((_EROWS, 128), jnp.float32),
                  jax.ShapeDtypeStruct((_EROWS, 128), jnp.float32)),
        mesh=mesh,
        scratch_types=[
            pltpu.VMEM((BATCH,), jnp.int32),
            pltpu.VMEM((_SEL + _L,), jnp.int32),
            pltpu.VMEM((_SEL + _L,), jnp.int32),
            pltpu.VMEM((_SEL + _L,), jnp.int32),
            pltpu.VMEM((_SEL + _L,), jnp.int32),
            pltpu.VMEM((_RING, 64, 128), jnp.float32),
            pltpu.VMEM((128, 128), jnp.float32),
            pltpu.VMEM((2, 64), jnp.int32),
            pltpu.VMEM((64 * 64,), jnp.float32),
            pltpu.SemaphoreType.DMA((_RING,)),
        ],
        compiler_params=pltpu.CompilerParams(needs_layout_passes=False),
    )
    ctail = center_table.T[:, VOCAB - 64:].reshape(64 * 64)
    xtail = context_table.T[:, VOCAB - 64:].reshape(64 * 64)
    e1, e2 = run1(cw, xw, center_table.T, context_table.T, ctail, xtail)
    run2 = pl.kernel(
        _phase2,
        out_type=jax.ShapeDtypeStruct((BATCH,), jnp.float32),
        mesh=mesh,
        scratch_types=[
            pltpu.VMEM((2, _C2, 128), jnp.float32),
            pltpu.VMEM((2, _C2, 128), jnp.float32),
            pltpu.VMEM((_RW,), jnp.float32),
            pltpu.SemaphoreType.DMA((2,)),
        ],
        compiler_params=pltpu.CompilerParams(needs_layout_passes=False),
    )
    return run2(e1, e2)
